# baseline (speedup 1.0000x reference)
import functools

import jax
import jax.numpy as jnp
from jax import lax
from jax.experimental import pallas as pl
from jax.experimental.pallas import tpu as pltpu
from jax.experimental.pallas import tpu_sc as plsc

N = 10000
E = 320000
F = 128
NC = 2
NS = 16
L = 16
HALF = 5000
NPH = 5120
GARB = 5056
ROWS_PS = NPH // NS
EPW = E // NS
CHUNK = 80
NCHUNK = EPW // CHUNK
NBATCH = 10
CPB = NCHUNK // NBATCH
GROUPS = CHUNK // L


def _gelu_exact(a):
    y = a * 0.7071067811865476
    ay = jnp.abs(y)
    t = 1.0 / (1.0 + 0.3275911 * ay)
    poly = t * (0.254829592 + t * (-0.284496736 + t * (
        1.421413741 + t * (-1.453152027 + t * 1.061405429))))
    erf = 1.0 - poly * jnp.exp(-(ay * ay))
    erf = jnp.where(y >= 0.0, erf, -erf)
    return 0.5 * a * (1.0 + erf)


def _dense_body(f_ref, w_ref, b_ref, awt_ref, z_ref, pq_ref):
    z = jnp.dot(f_ref[...], w_ref[...], preferred_element_type=jnp.float32)
    z = z + b_ref[...]
    z_ref[...] = z
    pq_ref[...] = lax.dot_general(
        awt_ref[...], z, (((1,), (1,)), ((), ())),
        preferred_element_type=jnp.float32)


_dense_call = pl.pallas_call(
    _dense_body,
    out_shape=[
        jax.ShapeDtypeStruct((N, F), jnp.float32),
        jax.ShapeDtypeStruct((2, N), jnp.float32),
    ],
)


_mesh = plsc.VectorSubcoreMesh(core_axis_name="c", subcore_axis_name="s")


@functools.partial(
    pl.kernel,
    out_type=[
        jax.ShapeDtypeStruct((NC, NPH, F), jnp.float32),
        jax.ShapeDtypeStruct((NC, NS, NPH), jnp.float32),
    ],
    mesh=_mesh,
    compiler_params=pltpu.CompilerParams(needs_layout_passes=False),
    scratch_types=[
        pltpu.VMEM((N,), jnp.float32),
        pltpu.VMEM((N,), jnp.float32),
        pltpu.VMEM((NPH,), jnp.float32),
        pltpu.VMEM((CPB, CHUNK), jnp.int32),
        pltpu.VMEM((CPB, CHUNK), jnp.int32),
        pltpu.VMEM((CHUNK,), jnp.float32),
        pltpu.VMEM((CHUNK, F), jnp.float32),
        pltpu.VMEM_SHARED((NPH, F), jnp.float32),
        pltpu.SemaphoreType.DMA,
    ],
)
def _sc_edge_kernel(pq_hbm, src_hbm, dst_hbm, z_hbm,
                    u_out, s_out,
                    p_v, q_v, s_v, src_v, dst_v, w_v, rows_v, u_sh,
                    sem):
    cid = lax.axis_index("c")
    sid = lax.axis_index("s")
    row0 = sid * ROWS_PS
    lo = cid * HALF
    hi = jnp.where(cid == 0, HALF, N)

    pltpu.sync_copy(pq_hbm.at[0], p_v)
    pltpu.sync_copy(pq_hbm.at[1], q_v)

    zero16 = jnp.zeros((L,), jnp.float32)

    def zs_body(i, carry):
        s_v[pl.ds(i * L, L)] = zero16
        return carry

    lax.fori_loop(0, NPH // L, zs_body, 0)

    def zr_body(r, carry):
        for j in range(F // L):
            rows_v[r, pl.ds(j * L, L)] = zero16
        return carry

    lax.fori_loop(0, CHUNK, zr_body, 0)
    for b in range(ROWS_PS // CHUNK):
        pltpu.sync_copy(rows_v, u_sh.at[pl.ds(row0 + b * CHUNK, CHUNK)])

    plsc.subcore_barrier()

    def batch_body(b, carry):
        pltpu.sync_copy(src_hbm.at[sid * NBATCH + b], src_v)
        pltpu.sync_copy(dst_hbm.at[sid * NBATCH + b], dst_v)
        lax.fori_loop(0, CPB, chunk_body, 0)
        return carry

    def chunk_body(c, carry):
        for g in range(GROUPS):
            s16 = src_v[c, pl.ds(g * L, L)]
            d16 = dst_v[c, pl.ds(g * L, L)]
            p16 = plsc.load_gather(p_v, [s16])
            q16 = plsc.load_gather(q_v, [d16])
            w16 = jnp.exp(_gelu_exact(p16 + q16))
            w_v[pl.ds(g * L, L)] = w16
            ok = (d16 >= lo) & (d16 < hi)
            l16 = jnp.where(ok, d16 - lo, GARB)
            dst_v[c, pl.ds(g * L, L)] = l16
            plsc.addupdate_scatter(s_v, [l16], w16)
        pltpu.async_copy(z_hbm.at[src_v.at[c]], rows_v, sem).wait()

        @plsc.parallel_loop(0, CHUNK, unroll=4)
        def scale(e):
            wb = plsc.load_gather(w_v, [jnp.full((L,), e, jnp.int32)])
            for j in range(F // L):
                rows_v[e, pl.ds(j * L, L)] = rows_v[e, pl.ds(j * L, L)] * wb

        pltpu.sync_copy(rows_v, u_sh.at[dst_v.at[c]], add=True)
        return carry

    lax.fori_loop(0, NBATCH, batch_body, 0)

    plsc.subcore_barrier()
    pltpu.sync_copy(u_sh.at[pl.ds(row0, ROWS_PS)],
                    u_out.at[cid, pl.ds(row0, ROWS_PS)])
    pltpu.sync_copy(s_v, s_out.at[cid, sid])


def _combine_body(u2_ref, sp_ref, h_ref):
    u = jnp.concatenate(
        [u2_ref[0, :HALF], u2_ref[1, :N - HALF]], axis=0)
    sp = jnp.sum(sp_ref[...], axis=1)
    s = jnp.concatenate([sp[0, :HALF], sp[1, :N - HALF]], axis=0)[:, None]
    safe = jnp.where(s > 0.0, s, 1.0)
    h_ref[...] = jnp.where(s > 0.0, u / safe, 0.0)


_combine_call = pl.pallas_call(
    _combine_body,
    out_shape=jax.ShapeDtypeStruct((N, F), jnp.float32),
)


@jax.jit
def kernel(feature, edge_index, W_out, b_out, attn_w):
    src = edge_index[0].reshape(NS * NBATCH, CPB, CHUNK)
    dst = edge_index[1].reshape(NS * NBATCH, CPB, CHUNK)
    awt = jnp.stack([attn_w[:F, 0], attn_w[F:, 0]])
    z, pq = _dense_call(feature, W_out, b_out.reshape(1, F), awt)
    u2, sp = _sc_edge_kernel(pq, src, dst, z)
    return _combine_call(u2, sp)


# speedup vs baseline: 1.6052x; 1.6052x over previous
import functools

import jax
import jax.numpy as jnp
from jax import lax
from jax.experimental import pallas as pl
from jax.experimental.pallas import tpu as pltpu
from jax.experimental.pallas import tpu_sc as plsc

N = 10000
E = 320000
F = 128
NC = 2
NS = 16
L = 16
HALF = 5000
NPH = 5120
GARB = 5056
ROWS_PS = NPH // NS
EPW = E // NS
CHUNK = 80
NCHUNK = EPW // CHUNK
NBATCH = 10
CPB = NCHUNK // NBATCH
GROUPS = CHUNK // L


def _gelu_exact(a):
    y = a * 0.7071067811865476
    ay = jnp.abs(y)
    t = 1.0 / (1.0 + 0.3275911 * ay)
    poly = t * (0.254829592 + t * (-0.284496736 + t * (
        1.421413741 + t * (-1.453152027 + t * 1.061405429))))
    erf = 1.0 - poly * jnp.exp(-(ay * ay))
    erf = jnp.where(y >= 0.0, erf, -erf)
    return 0.5 * a * (1.0 + erf)


def _dense_body(f_ref, w_ref, b_ref, awt_ref, z_ref, pq_ref):
    z = jnp.dot(f_ref[...], w_ref[...], preferred_element_type=jnp.float32)
    z = z + b_ref[...]
    z_ref[...] = z
    pq_ref[...] = lax.dot_general(
        awt_ref[...], z, (((1,), (1,)), ((), ())),
        preferred_element_type=jnp.float32)


_dense_call = pl.pallas_call(
    _dense_body,
    out_shape=[
        jax.ShapeDtypeStruct((N, F), jnp.float32),
        jax.ShapeDtypeStruct((2, N), jnp.float32),
    ],
)


_mesh = plsc.VectorSubcoreMesh(core_axis_name="c", subcore_axis_name="s")


@functools.partial(
    pl.kernel,
    out_type=[
        jax.ShapeDtypeStruct((NC, NPH, F), jnp.float32),
        jax.ShapeDtypeStruct((NC, NS, NPH), jnp.float32),
    ],
    mesh=_mesh,
    compiler_params=pltpu.CompilerParams(needs_layout_passes=False),
    scratch_types=[
        pltpu.VMEM((N,), jnp.float32),
        pltpu.VMEM((N,), jnp.float32),
        pltpu.VMEM((NPH,), jnp.float32),
        pltpu.VMEM((CPB, CHUNK), jnp.int32),
        pltpu.VMEM((CPB, CHUNK), jnp.int32),
        pltpu.VMEM((CHUNK,), jnp.float32),
        pltpu.VMEM((CHUNK, F), jnp.float32),
        pltpu.VMEM((CHUNK, F), jnp.float32),
        pltpu.VMEM_SHARED((NPH, F), jnp.float32),
        pltpu.SemaphoreType.DMA,
    ],
)
def _sc_edge_kernel(pq_hbm, src_hbm, dst_hbm, z_hbm,
                    u_out, s_out,
                    p_v, q_v, s_v, src_v, dst_v, w_v, rows_v, rows2_v, u_sh,
                    sem):
    cid = lax.axis_index("c")
    sid = lax.axis_index("s")
    row0 = sid * ROWS_PS
    lo = cid * HALF
    hi = jnp.where(cid == 0, HALF, N)

    pltpu.sync_copy(pq_hbm.at[0], p_v)
    pltpu.sync_copy(pq_hbm.at[1], q_v)

    zero16 = jnp.zeros((L,), jnp.float32)

    def zs_body(i, carry):
        s_v[pl.ds(i * L, L)] = zero16
        return carry

    lax.fori_loop(0, NPH // L, zs_body, 0)

    def zr_body(r, carry):
        for j in range(F // L):
            rows_v[r, pl.ds(j * L, L)] = zero16
        return carry

    lax.fori_loop(0, CHUNK, zr_body, 0)
    for b in range(ROWS_PS // CHUNK):
        pltpu.sync_copy(rows_v, u_sh.at[pl.ds(row0 + b * CHUNK, CHUNK)])

    plsc.subcore_barrier()

    def weights(c):
        for g in range(GROUPS):
            s16 = src_v[c, pl.ds(g * L, L)]
            d16 = dst_v[c, pl.ds(g * L, L)]
            p16 = plsc.load_gather(p_v, [s16])
            q16 = plsc.load_gather(q_v, [d16])
            w16 = jnp.exp(_gelu_exact(p16 + q16))
            w_v[pl.ds(g * L, L)] = w16
            ok = (d16 >= lo) & (d16 < hi)
            l16 = jnp.where(ok, d16 - lo, GARB)
            dst_v[c, pl.ds(g * L, L)] = l16
            plsc.addupdate_scatter(s_v, [l16], w16)

    def scale_scatter(c, rv):
        @plsc.parallel_loop(0, CHUNK, unroll=2)
        def scale(e):
            wb = plsc.load_gather(w_v, [jnp.full((L,), e, jnp.int32)])
            for j in range(F // L):
                rv[e, pl.ds(j * L, L)] = rv[e, pl.ds(j * L, L)] * wb

        pltpu.sync_copy(rv, u_sh.at[dst_v.at[c]], add=True)

    def batch_body(b, carry):
        pltpu.sync_copy(src_hbm.at[sid * NBATCH + b], src_v)
        pltpu.sync_copy(dst_hbm.at[sid * NBATCH + b], dst_v)
        pltpu.async_copy(z_hbm.at[src_v.at[0]], rows_v, sem)

        def pair_body(k, carry2):
            c0 = 2 * k
            pltpu.async_copy(z_hbm.at[src_v.at[c0 + 1]], rows2_v, sem)
            weights(c0)
            pltpu.make_async_copy(z_hbm.at[src_v.at[c0]], rows_v, sem).wait()
            scale_scatter(c0, rows_v)
            pltpu.async_copy(z_hbm.at[src_v.at[c0 + 2]], rows_v, sem)
            weights(c0 + 1)
            pltpu.make_async_copy(
                z_hbm.at[src_v.at[c0 + 1]], rows2_v, sem).wait()
            scale_scatter(c0 + 1, rows2_v)
            return carry2

        lax.fori_loop(0, (CPB - 1) // 2, pair_body, 0)
        weights(CPB - 1)
        pltpu.make_async_copy(
            z_hbm.at[src_v.at[CPB - 1]], rows_v, sem).wait()
        scale_scatter(CPB - 1, rows_v)
        return carry

    lax.fori_loop(0, NBATCH, batch_body, 0)

    plsc.subcore_barrier()
    pltpu.sync_copy(u_sh.at[pl.ds(row0, ROWS_PS)],
                    u_out.at[cid, pl.ds(row0, ROWS_PS)])
    pltpu.sync_copy(s_v, s_out.at[cid, sid])


def _combine_body(u2_ref, sp_ref, h_ref):
    u = jnp.concatenate(
        [u2_ref[0, :HALF], u2_ref[1, :N - HALF]], axis=0)
    sp = jnp.sum(sp_ref[...], axis=1)
    s = jnp.concatenate([sp[0, :HALF], sp[1, :N - HALF]], axis=0)[:, None]
    safe = jnp.where(s > 0.0, s, 1.0)
    h_ref[...] = jnp.where(s > 0.0, u / safe, 0.0)


_combine_call = pl.pallas_call(
    _combine_body,
    out_shape=jax.ShapeDtypeStruct((N, F), jnp.float32),
)


@jax.jit
def kernel(feature, edge_index, W_out, b_out, attn_w):
    src = edge_index[0].reshape(NS * NBATCH, CPB, CHUNK)
    dst = edge_index[1].reshape(NS * NBATCH, CPB, CHUNK)
    awt = jnp.stack([attn_w[:F, 0], attn_w[F:, 0]])
    z, pq = _dense_call(feature, W_out, b_out.reshape(1, F), awt)
    u2, sp = _sc_edge_kernel(pq, src, dst, z)
    return _combine_call(u2, sp)
